# baseline (speedup 1.0000x reference)
import functools

import jax
import jax.numpy as jnp
from jax import lax
from jax.experimental import pallas as pl
from jax.experimental.pallas import tpu as pltpu
from jax.experimental.pallas import tpu_sc as plsc

N_Q = 32
N_KV = 8192
D_HEAD = 128
MAX_OFFSET = 512

WIN_ROWS = 608
WIN_PER_TILE = WIN_ROWS // 16
HEAD = 576
CONST_LEN = 32
S_ROWS = WIN_ROWS + 16 * CONST_LEN
TAIL = N_KV - HEAD
N_FULL = TAIL // CONST_LEN
FIRE = 14


def _make_rpe():
    mesh = plsc.VectorSubcoreMesh(core_axis_name="c", subcore_axis_name="s")

    @functools.partial(
        pl.kernel,
        mesh=mesh,
        out_type=jax.ShapeDtypeStruct((N_Q, N_KV, D_HEAD), jnp.float32),
        scratch_types=[
            pltpu.VMEM_SHARED((S_ROWS, D_HEAD), jnp.float32),
            pltpu.SemaphoreType.DMA,
            pltpu.SemaphoreType.DMA,
        ],
    )
    def rpe(table_hbm, out_hbm, s_ref, bsem, wsem):
        c = lax.axis_index("c")
        s = lax.axis_index("s")
        q = c * 16 + s

        u0 = s * WIN_PER_TILE
        cstart = WIN_ROWS + s * CONST_LEN
        rows = [(jnp.maximum(543 - (u0 + r), 0), u0 + r)
                for r in range(WIN_PER_TILE)]
        rows += [(0, cstart + r) for r in range(CONST_LEN)]
        for base in range(0, len(rows), FIRE):
            burst = [
                pltpu.async_copy(table_hbm.at[src], s_ref.at[dst], bsem)
                for src, dst in rows[base:base + FIRE]
            ]
            for cp in burst:
                cp.wait()

        plsc.subcore_barrier()

        writes = [
            pltpu.async_copy(
                s_ref.at[pl.ds(31 - q, HEAD)],
                out_hbm.at[q, pl.ds(0, HEAD)],
                wsem,
            )
        ]
        const_src = s_ref.at[pl.ds(cstart, CONST_LEN)]
        for i in range(N_FULL):
            writes.append(
                pltpu.async_copy(
                    const_src,
                    out_hbm.at[q, pl.ds(HEAD + i * CONST_LEN, CONST_LEN)],
                    wsem,
                )
            )
        for cp in writes:
            cp.wait()

    return rpe


_rpe = _make_rpe()


def kernel(n_q, n_kv, pos_emb):
    del n_q, n_kv
    return _rpe(pos_emb)


# speedup vs baseline: 1.5205x; 1.5205x over previous
import functools

import jax
import jax.numpy as jnp
from jax import lax
from jax.experimental import pallas as pl
from jax.experimental.pallas import tpu as pltpu
from jax.experimental.pallas import tpu_sc as plsc

N_Q = 32
N_KV = 8192
D_HEAD = 128
MAX_OFFSET = 512

WIN_ROWS = 608
WIN_PER_TILE = WIN_ROWS // 16
HEAD = 576
CONST_START = 608
CONST_LEN = 32
S_ROWS = CONST_START + CONST_LEN
TAIL = N_KV - HEAD
N_FULL = TAIL // CONST_LEN
FIRE = 13


def _make_rpe():
    mesh = plsc.VectorSubcoreMesh(core_axis_name="c", subcore_axis_name="s")

    @functools.partial(
        pl.kernel,
        mesh=mesh,
        out_type=jax.ShapeDtypeStruct((N_Q, N_KV, D_HEAD), jnp.float32),
        scratch_types=[
            pltpu.VMEM_SHARED((S_ROWS, D_HEAD), jnp.float32),
            pltpu.SemaphoreType.DMA,
            pltpu.SemaphoreType.DMA,
        ],
    )
    def rpe(table_hbm, out_hbm, s_ref, bsem, wsem):
        c = lax.axis_index("c")
        s = lax.axis_index("s")
        q = c * 16 + s

        cb = [
            pltpu.async_copy(table_hbm.at[0], s_ref.at[CONST_START + 2 * s + r],
                             bsem)
            for r in range(2)
        ]
        for cp in cb:
            cp.wait()
        plsc.subcore_barrier()

        const_src = s_ref.at[pl.ds(CONST_START, CONST_LEN)]
        writes = [
            pltpu.async_copy(
                const_src,
                out_hbm.at[q, pl.ds(HEAD + i * CONST_LEN, CONST_LEN)],
                wsem,
            )
            for i in range(N_FULL)
        ]

        u0 = s * WIN_PER_TILE
        for base in range(0, WIN_PER_TILE, FIRE):
            burst = [
                pltpu.async_copy(
                    table_hbm.at[jnp.maximum(543 - (u0 + base + r), 0)],
                    s_ref.at[u0 + base + r],
                    bsem,
                )
                for r in range(min(FIRE, WIN_PER_TILE - base))
            ]
            for cp in burst:
                cp.wait()
        plsc.subcore_barrier()

        writes.append(
            pltpu.async_copy(
                s_ref.at[pl.ds(31 - q, HEAD)],
                out_hbm.at[q, pl.ds(0, HEAD)],
                wsem,
            )
        )
        for cp in writes:
            cp.wait()

    return rpe


_rpe = _make_rpe()


def kernel(n_q, n_kv, pos_emb):
    del n_q, n_kv
    return _rpe(pos_emb)
